# Initial kernel scaffold; baseline (speedup 1.0000x reference)
import jax
import jax.numpy as jnp
from jax.experimental import pallas as pl


def kernel(X, edge_idx, C):
    raise NotImplementedError("write your pallas kernel here")


# speedup vs baseline: 8.2197x; 8.2197x over previous
import functools

import jax
import jax.numpy as jnp
from jax import lax
from jax.experimental import pallas as pl
from jax.experimental.pallas import tpu as pltpu
from jax.experimental.pallas import tpu_sc as plsc

N = 10000
K = 64
D12 = 12
DH = 3 * D12
DHP = 48
NW = 32
EPW = N * K // NW
BB = 80
NB = EPW // BB
MV = EPW // 16
NBLK = 200


def _tables_body(x_ref, c_ref, h_ref, m_ref):
    x = x_ref[...]
    zf = jnp.zeros((1, D12), jnp.float32)
    left = jnp.concatenate([zf, x[:-1, :]], axis=0)
    right = jnp.concatenate([x[1:, :], zf], axis=0)
    h_ref[...] = jnp.concatenate(
        [left, x, right, jnp.zeros((N, DHP - DH), jnp.float32)], axis=1)

    c = c_ref[...]
    zi = jnp.zeros((1, 1), jnp.int32)
    cl = jnp.concatenate([zi, c[:-1, :]], axis=0)
    cr = jnp.concatenate([c[1:, :], zi], axis=0)
    m_ref[...] = ((cl > 0) & (cl == c) & (cl == cr)).astype(jnp.float32)


_tables = pl.pallas_call(
    _tables_body,
    out_shape=[
        jax.ShapeDtypeStruct((N, DHP), jnp.float32),
        jax.ShapeDtypeStruct((N, 1), jnp.float32),
    ],
)


def _sc_body(htab, mtab, edges, xg, mout,
             bidx_v, row_v, mtab_v, edge_v, mask_v, sem):
    w = lax.axis_index("s") * 2 + lax.axis_index("c")

    pltpu.sync_copy(edges.at[w], edge_v)
    pltpu.sync_copy(mtab, mtab_v)

    lanes = lax.iota(jnp.int32, 16)

    def mask_body(b, carry):
        for u in range(BB // 16):
            e = edge_v[b, pl.ds(u * 16, 16)]
            nn = jnp.right_shift(lanes + (w * EPW + b * BB + u * 16), 6)
            mn = plsc.load_gather(mtab_v, [nn])
            me = plsc.load_gather(mtab_v, [e])
            mask_v[pl.ds(b * BB + u * 16, 16)] = mn * me
        return carry

    lax.fori_loop(0, NB, mask_body, 0)
    pltpu.sync_copy(mask_v, mout.at[w])

    def gather_body(b, carry):
        pltpu.sync_copy(edges.at[w, b], bidx_v)
        pltpu.async_copy(htab.at[bidx_v], row_v, sem).wait()
        pltpu.sync_copy(row_v, xg.at[w, b])
        return carry

    lax.fori_loop(0, NB, gather_body, 0)


_sc_gather = pl.kernel(
    _sc_body,
    out_type=[
        jax.ShapeDtypeStruct((NW, NB, BB, DHP), jnp.float32),
        jax.ShapeDtypeStruct((NW, EPW), jnp.float32),
    ],
    mesh=plsc.VectorSubcoreMesh(core_axis_name="c", subcore_axis_name="s"),
    compiler_params=pltpu.CompilerParams(
        needs_layout_passes=False, use_tc_tiling_on_sc=False),
    scratch_types=[
        pltpu.VMEM((BB,), jnp.int32),
        pltpu.VMEM((BB, DHP), jnp.float32),
        pltpu.VMEM((N,), jnp.float32),
        pltpu.VMEM((NB, BB), jnp.int32),
        pltpu.VMEM((EPW,), jnp.float32),
        pltpu.SemaphoreType.DMA,
    ],
)


def _asm_body(h_ref, g_ref, o_ref):
    h = h_ref[...][:, :DH]
    g = g_ref[...][:, :, :DH]
    hb = jnp.broadcast_to(h[:, None, :], (NBLK, K, DH))
    o_ref[...] = jnp.concatenate([hb, g], axis=-1)


_assemble = pl.pallas_call(
    _asm_body,
    grid=(N // NBLK,),
    in_specs=[
        pl.BlockSpec((NBLK, DHP), lambda i: (i, 0)),
        pl.BlockSpec((NBLK, K, DHP), lambda i: (i, 0, 0)),
    ],
    out_specs=pl.BlockSpec((NBLK, K, 2 * DH), lambda i: (i, 0, 0)),
    out_shape=jax.ShapeDtypeStruct((N, K, 2 * DH), jnp.float32),
)


def kernel(X, edge_idx, C):
    xf = X.reshape(N, D12)
    c2 = C.reshape(N, 1)
    htab, mtab = _tables(xf, c2)

    edge_slabs = edge_idx.reshape(NW, NB, BB)
    xg, mout = _sc_gather(htab, mtab.reshape(N), edge_slabs)

    xout = _assemble(htab, xg.reshape(N, K, DHP))

    X_ij = xout.reshape(1, N, K, 2 * DH // 3, 3)
    mask_ij = mout.reshape(1, N, K, 1)
    return X_ij, mask_ij


# speedup vs baseline: 9.7957x; 1.1917x over previous
import functools

import jax
import jax.numpy as jnp
from jax import lax
from jax.experimental import pallas as pl
from jax.experimental.pallas import tpu as pltpu
from jax.experimental.pallas import tpu_sc as plsc

N = 10000
K = 64
D12 = 12
DH = 3 * D12
DHP = 48
NW = 32
EPW = N * K // NW
BB = 400
NB = EPW // BB
MV = EPW // 16
NBLK = 200


def _tables_body(x_ref, c_ref, h_ref, m_ref):
    x = x_ref[...]
    zf = jnp.zeros((1, D12), jnp.float32)
    left = jnp.concatenate([zf, x[:-1, :]], axis=0)
    right = jnp.concatenate([x[1:, :], zf], axis=0)
    h_ref[...] = jnp.concatenate(
        [left, x, right, jnp.zeros((N, DHP - DH), jnp.float32)], axis=1)

    c = c_ref[...]
    zi = jnp.zeros((1, 1), jnp.int32)
    cl = jnp.concatenate([zi, c[:-1, :]], axis=0)
    cr = jnp.concatenate([c[1:, :], zi], axis=0)
    m_ref[...] = ((cl > 0) & (cl == c) & (cl == cr)).astype(jnp.float32)


_tables = pl.pallas_call(
    _tables_body,
    out_shape=[
        jax.ShapeDtypeStruct((N, DHP), jnp.float32),
        jax.ShapeDtypeStruct((N, 1), jnp.float32),
    ],
)


def _sc_body(htab, mtab, edges, xg, mout,
             bidx_v, row_v, mtab_v, edge_v, mask_v, sem):
    w = lax.axis_index("s") * 2 + lax.axis_index("c")

    pltpu.sync_copy(edges.at[w], edge_v)
    pltpu.sync_copy(mtab, mtab_v)

    lanes = lax.iota(jnp.int32, 16)

    def mask_body(b, carry):
        for u in range(BB // 16):
            e = edge_v[b, pl.ds(u * 16, 16)]
            nn = jnp.right_shift(lanes + (w * EPW + b * BB + u * 16), 6)
            mn = plsc.load_gather(mtab_v, [nn])
            me = plsc.load_gather(mtab_v, [e])
            mask_v[pl.ds(b * BB + u * 16, 16)] = mn * me
        return carry

    lax.fori_loop(0, NB, mask_body, 0)
    pltpu.sync_copy(mask_v, mout.at[w])

    def gather_body(b, carry):
        pltpu.sync_copy(edges.at[w, b], bidx_v)
        pltpu.async_copy(htab.at[bidx_v], row_v, sem).wait()
        pltpu.sync_copy(row_v, xg.at[w, b])
        return carry

    lax.fori_loop(0, NB, gather_body, 0)


_sc_gather = pl.kernel(
    _sc_body,
    out_type=[
        jax.ShapeDtypeStruct((NW, NB, BB, DHP), jnp.float32),
        jax.ShapeDtypeStruct((NW, EPW), jnp.float32),
    ],
    mesh=plsc.VectorSubcoreMesh(core_axis_name="c", subcore_axis_name="s"),
    compiler_params=pltpu.CompilerParams(
        needs_layout_passes=False, use_tc_tiling_on_sc=False),
    scratch_types=[
        pltpu.VMEM((BB,), jnp.int32),
        pltpu.VMEM((BB, DHP), jnp.float32),
        pltpu.VMEM((N,), jnp.float32),
        pltpu.VMEM((NB, BB), jnp.int32),
        pltpu.VMEM((EPW,), jnp.float32),
        pltpu.SemaphoreType.DMA,
    ],
)


def _asm_body(h_ref, g_ref, o_ref):
    h = h_ref[...][:, :DH]
    g = g_ref[...][:, :, :DH]
    hb = jnp.broadcast_to(h[:, None, :], (NBLK, K, DH))
    o_ref[...] = jnp.concatenate([hb, g], axis=-1)


_assemble = pl.pallas_call(
    _asm_body,
    grid=(N // NBLK,),
    in_specs=[
        pl.BlockSpec((NBLK, DHP), lambda i: (i, 0)),
        pl.BlockSpec((NBLK, K, DHP), lambda i: (i, 0, 0)),
    ],
    out_specs=pl.BlockSpec((NBLK, K, 2 * DH), lambda i: (i, 0, 0)),
    out_shape=jax.ShapeDtypeStruct((N, K, 2 * DH), jnp.float32),
)


def kernel(X, edge_idx, C):
    xf = X.reshape(N, D12)
    c2 = C.reshape(N, 1)
    htab, mtab = _tables(xf, c2)

    edge_slabs = edge_idx.reshape(NW, NB, BB)
    xg, mout = _sc_gather(htab, mtab.reshape(N), edge_slabs)

    xout = _assemble(htab, xg.reshape(N, K, DHP))

    X_ij = xout.reshape(1, N, K, 2 * DH // 3, 3)
    mask_ij = mout.reshape(1, N, K, 1)
    return X_ij, mask_ij


# speedup vs baseline: 9.8017x; 1.0006x over previous
import functools

import jax
import jax.numpy as jnp
from jax import lax
from jax.experimental import pallas as pl
from jax.experimental.pallas import tpu as pltpu
from jax.experimental.pallas import tpu_sc as plsc

N = 10000
K = 64
D12 = 12
DH = 3 * D12
DHP = 48
NW = 32
EPW = N * K // NW
BB = 400
NB = EPW // BB
MV = EPW // 16
NBLK = 200


def _tables_body(x_ref, c_ref, h_ref, m_ref):
    x = x_ref[...]
    zf = jnp.zeros((1, D12), jnp.float32)
    left = jnp.concatenate([zf, x[:-1, :]], axis=0)
    right = jnp.concatenate([x[1:, :], zf], axis=0)
    h_ref[...] = jnp.concatenate(
        [left, x, right, jnp.zeros((N, DHP - DH), jnp.float32)], axis=1)

    c = c_ref[...]
    zi = jnp.zeros((1, 1), jnp.int32)
    cl = jnp.concatenate([zi, c[:-1, :]], axis=0)
    cr = jnp.concatenate([c[1:, :], zi], axis=0)
    m_ref[...] = ((cl > 0) & (cl == c) & (cl == cr)).astype(jnp.float32)


_tables = pl.pallas_call(
    _tables_body,
    out_shape=[
        jax.ShapeDtypeStruct((N, DHP), jnp.float32),
        jax.ShapeDtypeStruct((N, 1), jnp.float32),
    ],
)


def _sc_body(htab, mtab, edges, xg, mout,
             bidx_v, row_v, mtab_v, edge_v, mask_v, sem):
    w = lax.axis_index("s") * 2 + lax.axis_index("c")

    pltpu.sync_copy(edges.at[w], edge_v)
    pltpu.sync_copy(mtab, mtab_v)

    lanes = lax.iota(jnp.int32, 16)

    def mask_body(b, carry):
        for u in range(BB // 16):
            e = edge_v[b, pl.ds(u * 16, 16)]
            nn = jnp.right_shift(lanes + (w * EPW + b * BB + u * 16), 6)
            mn = plsc.load_gather(mtab_v, [nn])
            me = plsc.load_gather(mtab_v, [e])
            mask_v[pl.ds(b * BB + u * 16, 16)] = mn * me
        return carry

    lax.fori_loop(0, NB, mask_body, 0)
    pltpu.sync_copy(mask_v, mout.at[w])

    def gather_body(b, carry):
        pltpu.sync_copy(edges.at[w, b], bidx_v)
        pltpu.async_copy(htab.at[bidx_v], row_v, sem).wait()
        pltpu.sync_copy(row_v, xg.at[w, b])
        return carry

    lax.fori_loop(0, NB, gather_body, 0)


_sc_gather = pl.kernel(
    _sc_body,
    out_type=[
        jax.ShapeDtypeStruct((NW, NB, BB, DHP), jnp.float32),
        jax.ShapeDtypeStruct((NW, EPW), jnp.float32),
    ],
    mesh=plsc.VectorSubcoreMesh(core_axis_name="c", subcore_axis_name="s"),
    compiler_params=pltpu.CompilerParams(
        needs_layout_passes=False, use_tc_tiling_on_sc=False),
    scratch_types=[
        pltpu.VMEM((BB,), jnp.int32),
        pltpu.VMEM((BB, DHP), jnp.float32),
        pltpu.VMEM((N,), jnp.float32),
        pltpu.VMEM((NB, BB), jnp.int32),
        pltpu.VMEM((EPW,), jnp.float32),
        pltpu.SemaphoreType.DMA,
    ],
)


def _asm_inner(h_ref, g_ref, o_ref):
    h = h_ref[...][:, :DH]
    g = g_ref[...][:, :, :DH]
    hb = jnp.broadcast_to(h[:, None, :], (NBLK, K, DH))
    o_ref[...] = jnp.concatenate([hb, g], axis=-1)


def _asm_outer(h_hbm, g_hbm, o_hbm):
    pipeline = pltpu.emit_pipeline(
        _asm_inner,
        grid=(N // NBLK,),
        in_specs=[
            pl.BlockSpec((NBLK, DHP), lambda i: (i, 0)),
            pl.BlockSpec((NBLK, K, DHP), lambda i: (i, 0, 0)),
        ],
        out_specs=[pl.BlockSpec((NBLK, K, 2 * DH), lambda i: (i, 0, 0))],
    )
    pipeline(h_hbm, g_hbm, o_hbm)


_assemble = pl.pallas_call(
    _asm_outer,
    in_specs=[
        pl.BlockSpec(memory_space=pl.ANY),
        pl.BlockSpec(memory_space=pl.ANY),
    ],
    out_specs=pl.BlockSpec(memory_space=pl.ANY),
    out_shape=jax.ShapeDtypeStruct((N, K, 2 * DH), jnp.float32),
)


def kernel(X, edge_idx, C):
    xf = X.reshape(N, D12)
    c2 = C.reshape(N, 1)
    htab, mtab = _tables(xf, c2)

    edge_slabs = edge_idx.reshape(NW, NB, BB)
    xg, mout = _sc_gather(htab, mtab.reshape(N), edge_slabs)

    xout = _assemble(htab, xg.reshape(N, K, DHP))

    X_ij = xout.reshape(1, N, K, 2 * DH // 3, 3)
    mask_ij = mout.reshape(1, N, K, 1)
    return X_ij, mask_ij


# speedup vs baseline: 12.5322x; 1.2786x over previous
import functools

import jax
import jax.numpy as jnp
from jax import lax
from jax.experimental import pallas as pl
from jax.experimental.pallas import tpu as pltpu
from jax.experimental.pallas import tpu_sc as plsc

N = 10000
K = 64
D12 = 12
DH = 3 * D12
DHP = 48
NW = 32
EPW = N * K // NW
BB = 400
NB = EPW // BB
MV = EPW // 16
NBLK = 200


def _tables_body(x_ref, c_ref, h_ref, m_ref):
    x = x_ref[...]
    zf = jnp.zeros((1, D12), jnp.float32)
    left = jnp.concatenate([zf, x[:-1, :]], axis=0)
    right = jnp.concatenate([x[1:, :], zf], axis=0)
    h_ref[...] = jnp.concatenate(
        [left, x, right, jnp.zeros((N, DHP - DH), jnp.float32)], axis=1)

    c = c_ref[...]
    zi = jnp.zeros((1, 1), jnp.int32)
    cl = jnp.concatenate([zi, c[:-1, :]], axis=0)
    cr = jnp.concatenate([c[1:, :], zi], axis=0)
    m_ref[...] = ((cl > 0) & (cl == c) & (cl == cr)).astype(jnp.float32)


_tables = pl.pallas_call(
    _tables_body,
    out_shape=[
        jax.ShapeDtypeStruct((N, DHP), jnp.float32),
        jax.ShapeDtypeStruct((N, 1), jnp.float32),
    ],
)


def _sc_body(htab, mtab, edges, xg, mout,
             bidx_v, row_v, mtab_v, edge_v, mask_v, sem):
    w = lax.axis_index("s") * 2 + lax.axis_index("c")

    pltpu.sync_copy(edges.at[w], edge_v)
    pltpu.sync_copy(mtab, mtab_v)

    lanes = lax.iota(jnp.int32, 16)

    def mask_body(b, carry):
        for u in range(BB // 16):
            e = edge_v[b, pl.ds(u * 16, 16)]
            nn = jnp.right_shift(lanes + (w * EPW + b * BB + u * 16), 6)
            mn = plsc.load_gather(mtab_v, [nn])
            me = plsc.load_gather(mtab_v, [e])
            mask_v[pl.ds(b * BB + u * 16, 16)] = mn * me
        return carry

    lax.fori_loop(0, NB, mask_body, 0)
    pltpu.sync_copy(mask_v, mout.at[w])

    def gather_body(b, carry):
        pltpu.sync_copy(edges.at[w, b], bidx_v)
        pltpu.async_copy(htab.at[bidx_v], row_v, sem).wait()
        pltpu.sync_copy(row_v,
                        xg.at[pl.ds(w * EPW + b * BB, BB), pl.ds(0, DHP)])
        return carry

    lax.fori_loop(0, NB, gather_body, 0)


_sc_gather = pl.kernel(
    _sc_body,
    out_type=[
        jax.ShapeDtypeStruct((N * K, 128), jnp.float32),
        jax.ShapeDtypeStruct((NW, EPW), jnp.float32),
    ],
    mesh=plsc.VectorSubcoreMesh(core_axis_name="c", subcore_axis_name="s"),
    compiler_params=pltpu.CompilerParams(
        needs_layout_passes=False, use_tc_tiling_on_sc=False),
    scratch_types=[
        pltpu.VMEM((BB,), jnp.int32),
        pltpu.VMEM((BB, DHP), jnp.float32),
        pltpu.VMEM((N,), jnp.float32),
        pltpu.VMEM((NB, BB), jnp.int32),
        pltpu.VMEM((EPW,), jnp.float32),
        pltpu.SemaphoreType.DMA,
    ],
)


def _asm_body(h_ref, g_ref, o_ref):
    h = h_ref[...][:, :DH]
    g = g_ref[...][:, :, :DH]
    hb = jnp.broadcast_to(h[:, None, :], (NBLK, K, DH))
    o_ref[...] = jnp.concatenate([hb, g], axis=-1)


_assemble = pl.pallas_call(
    _asm_body,
    grid=(N // NBLK,),
    in_specs=[
        pl.BlockSpec((NBLK, DHP), lambda i: (i, 0)),
        pl.BlockSpec((NBLK, K, 128), lambda i: (i, 0, 0)),
    ],
    out_specs=pl.BlockSpec((NBLK, K, 2 * DH), lambda i: (i, 0, 0)),
    out_shape=jax.ShapeDtypeStruct((N, K, 2 * DH), jnp.float32),
)


def kernel(X, edge_idx, C):
    xf = X.reshape(N, D12)
    c2 = C.reshape(N, 1)
    htab, mtab = _tables(xf, c2)

    edge_slabs = edge_idx.reshape(NW, NB, BB)
    xg, mout = _sc_gather(htab, mtab.reshape(N), edge_slabs)

    xout = _assemble(htab, xg.reshape(N, K, 128))

    X_ij = xout.reshape(1, N, K, 2 * DH // 3, 3)
    mask_ij = mout.reshape(1, N, K, 1)
    return X_ij, mask_ij


# speedup vs baseline: 13.5224x; 1.0790x over previous
import functools

import jax
import jax.numpy as jnp
from jax import lax
from jax.experimental import pallas as pl
from jax.experimental.pallas import tpu as pltpu
from jax.experimental.pallas import tpu_sc as plsc

N = 10000
K = 64
D12 = 12
DH = 3 * D12
DHP = 48
NW = 32
EPW = N * K // NW
BB = 400
NB = EPW // BB
MV = EPW // 16
NBLK = 200


def _tables_body(x_ref, c_ref, h_ref, m_ref):
    x = x_ref[...]
    zf = jnp.zeros((1, D12), jnp.float32)
    left = jnp.concatenate([zf, x[:-1, :]], axis=0)
    right = jnp.concatenate([x[1:, :], zf], axis=0)
    h_ref[...] = jnp.concatenate(
        [left, x, right, jnp.zeros((N, DHP - DH), jnp.float32)], axis=1)

    c = c_ref[...]
    zi = jnp.zeros((1, 1), jnp.int32)
    cl = jnp.concatenate([zi, c[:-1, :]], axis=0)
    cr = jnp.concatenate([c[1:, :], zi], axis=0)
    m_ref[...] = ((cl > 0) & (cl == c) & (cl == cr)).astype(jnp.float32)


_tables = pl.pallas_call(
    _tables_body,
    out_shape=[
        jax.ShapeDtypeStruct((N, DHP), jnp.float32),
        jax.ShapeDtypeStruct((N, 1), jnp.float32),
    ],
)


def _sc_body(htab, mtab, edges, xg, mout,
             bidx0, bidx1, row0, row1, mtab_v, edge_v, mask_v,
             sg0, sg1, sw0, sw1, si0, si1):
    w = lax.axis_index("s") * 2 + lax.axis_index("c")
    slots = ((bidx0, row0, sg0, sw0, si0), (bidx1, row1, sg1, sw1, si1))

    pltpu.sync_copy(edges.at[w], edge_v)
    pltpu.sync_copy(mtab, mtab_v)

    lanes = lax.iota(jnp.int32, 16)

    def mask_body(b, carry):
        for u in range(BB // 16):
            e = edge_v[b, pl.ds(u * 16, 16)]
            nn = jnp.right_shift(lanes + (w * EPW + b * BB + u * 16), 6)
            mn = plsc.load_gather(mtab_v, [nn])
            me = plsc.load_gather(mtab_v, [e])
            mask_v[pl.ds(b * BB + u * 16, 16)] = mn * me
        return carry

    lax.fori_loop(0, NB, mask_body, 0)
    pltpu.sync_copy(mask_v, mout.at[w])

    def dst(b):
        return xg.at[pl.ds(w * EPW + b * BB, BB), pl.ds(0, DHP)]

    pltpu.async_copy(edges.at[w, 0], bidx0, si0)
    pltpu.async_copy(edges.at[w, 1], bidx1, si1)

    def gather_body(t, carry):
        for s in (0, 1):
            b = 2 * t + s
            bi, ro, sg, sw, si = slots[s]

            @pl.when(b >= 2)
            def _wait_write():
                pltpu.make_async_copy(ro, dst(b - 2), sw).wait()

            pltpu.make_async_copy(edges.at[w, b], bi, si).wait()
            pltpu.async_copy(htab.at[bi], ro, sg)
        for s in (0, 1):
            b = 2 * t + s
            bi, ro, sg, sw, si = slots[s]
            pltpu.make_async_copy(htab.at[bi], ro, sg).wait()
            pltpu.async_copy(ro, dst(b), sw)

            @pl.when(b + 2 < NB)
            def _prefetch_idx():
                pltpu.async_copy(edges.at[w, b + 2], bi, si)

        return carry

    lax.fori_loop(0, NB // 2, gather_body, 0)
    pltpu.make_async_copy(row0, dst(NB - 2), sw0).wait()
    pltpu.make_async_copy(row1, dst(NB - 1), sw1).wait()


_sc_gather = pl.kernel(
    _sc_body,
    out_type=[
        jax.ShapeDtypeStruct((N * K, 128), jnp.float32),
        jax.ShapeDtypeStruct((NW, EPW), jnp.float32),
    ],
    mesh=plsc.VectorSubcoreMesh(core_axis_name="c", subcore_axis_name="s"),
    compiler_params=pltpu.CompilerParams(
        needs_layout_passes=False, use_tc_tiling_on_sc=False),
    scratch_types=[
        pltpu.VMEM((BB,), jnp.int32),
        pltpu.VMEM((BB,), jnp.int32),
        pltpu.VMEM((BB, DHP), jnp.float32),
        pltpu.VMEM((BB, DHP), jnp.float32),
        pltpu.VMEM((N,), jnp.float32),
        pltpu.VMEM((NB, BB), jnp.int32),
        pltpu.VMEM((EPW,), jnp.float32),
        pltpu.SemaphoreType.DMA,
        pltpu.SemaphoreType.DMA,
        pltpu.SemaphoreType.DMA,
        pltpu.SemaphoreType.DMA,
        pltpu.SemaphoreType.DMA,
        pltpu.SemaphoreType.DMA,
    ],
)


def _asm_body(h_ref, g_ref, o_ref):
    h = h_ref[...][:, :DH]
    g = g_ref[...][:, :, :DH]
    hb = jnp.broadcast_to(h[:, None, :], (NBLK, K, DH))
    o_ref[...] = jnp.concatenate([hb, g], axis=-1)


_assemble = pl.pallas_call(
    _asm_body,
    grid=(N // NBLK,),
    in_specs=[
        pl.BlockSpec((NBLK, DHP), lambda i: (i, 0)),
        pl.BlockSpec((NBLK, K, 128), lambda i: (i, 0, 0)),
    ],
    out_specs=pl.BlockSpec((NBLK, K, 2 * DH), lambda i: (i, 0, 0)),
    out_shape=jax.ShapeDtypeStruct((N, K, 2 * DH), jnp.float32),
)


def kernel(X, edge_idx, C):
    xf = X.reshape(N, D12)
    c2 = C.reshape(N, 1)
    htab, mtab = _tables(xf, c2)

    edge_slabs = edge_idx.reshape(NW, NB, BB)
    xg, mout = _sc_gather(htab, mtab.reshape(N), edge_slabs)

    xout = _assemble(htab, xg.reshape(N, K, 128))

    X_ij = xout.reshape(1, N, K, 2 * DH // 3, 3)
    mask_ij = mout.reshape(1, N, K, 1)
    return X_ij, mask_ij
